# baseline (speedup 1.0000x reference)
import functools

import jax
import jax.numpy as jnp
from jax import lax
from jax.experimental import pallas as pl
from jax.experimental.pallas import tpu as pltpu
from jax.experimental.pallas import tpu_sc as plsc

_N_CODES = 512
_CODE_DIM = 64
_H_TILE = 16

_NC = 2
_NS = 16
_NW = _NC * _NS
_TOK_PER_W = 2048
_CHUNK = 256


def _vq_tc_body(z_ref, cb_ref, idx_ref, acc_ref):
    x = z_ref[0].reshape(_CODE_DIM, _H_TILE * 64)
    cb = cb_ref[...]
    cnorm = jnp.sum(cb * cb, axis=1)
    znorm = jnp.sum(x * x, axis=0)
    s2 = lax.dot_general(cb * (-2.0), x, (((1,), (0,)), ((), ())),
                         preferred_element_type=jnp.float32)
    dist = (znorm[None, :] + s2) + cnorm[:, None]
    m = jnp.min(dist, axis=0)
    kiota = lax.broadcasted_iota(jnp.int32, dist.shape, 0)
    idx = jnp.min(jnp.where(dist == m[None, :], kiota, _N_CODES), axis=0)
    idx_ref[0, 0, :] = idx

    @pl.when((pl.program_id(0) == 0) & (pl.program_id(1) == 0))
    def _():
        acc_ref[0, 0] = 0.0

    acc_ref[0, 0] += jnp.sum(m)


def _sc_lookup_body(cb_hbm, idx_hbm, out_hbm, cb_v, idx_v, buf_v, sem):
    w = lax.axis_index("s") * _NC + lax.axis_index("c")
    b = w // 2
    hw0 = (w % 2) * _TOK_PER_W
    pltpu.sync_copy(cb_hbm, cb_v)
    pltpu.sync_copy(idx_hbm.at[w], idx_v)

    n_chunks = _TOK_PER_W // _CHUNK

    def outer(o, carry):
        for k in range(2):
            ch = o * 2 + k

            @pl.when(o > 0)
            def _():
                pltpu.make_async_copy(
                    buf_v.at[k], out_hbm.at[b, :, pl.ds(0, _CHUNK)], sem
                ).wait()

            def inner(l, carry2):
                i16 = idx_v[pl.ds(ch * _CHUNK + l * 16, 16)]
                base = i16 * _CODE_DIM
                for c in range(_CODE_DIM):
                    vals = plsc.load_gather(cb_v, [base + c])
                    buf_v[k, c, pl.ds(l * 16, 16)] = vals
                return carry2

            lax.fori_loop(0, _CHUNK // 16, inner, 0)
            pltpu.async_copy(
                buf_v.at[k],
                out_hbm.at[b, :, pl.ds(hw0 + ch * _CHUNK, _CHUNK)],
                sem,
            )
        return carry

    lax.fori_loop(0, n_chunks // 2, outer, 0)
    for k in range(2):
        pltpu.make_async_copy(
            buf_v.at[k], out_hbm.at[b, :, pl.ds(0, _CHUNK)], sem
        ).wait()


@jax.jit
def kernel(z, codebook):
    B, C, H, W = z.shape
    nh = H // _H_TILE
    idx3, acc = pl.pallas_call(
        _vq_tc_body,
        grid=(B, nh),
        in_specs=[
            pl.BlockSpec((1, C, _H_TILE, W), lambda b, h: (b, 0, h, 0)),
            pl.BlockSpec((_N_CODES, _CODE_DIM), lambda b, h: (0, 0)),
        ],
        out_specs=[
            pl.BlockSpec((1, 1, _H_TILE * W), lambda b, h, nh=nh: (b * nh + h, 0, 0)),
            pl.BlockSpec(memory_space=pltpu.SMEM),
        ],
        out_shape=[
            jax.ShapeDtypeStruct((B * nh, 1, _H_TILE * W), jnp.int32),
            jax.ShapeDtypeStruct((1, 1), jnp.float32),
        ],
    )(z, codebook)

    sc_lookup = pl.kernel(
        _sc_lookup_body,
        out_type=jax.ShapeDtypeStruct((B, C, H * W), jnp.float32),
        mesh=plsc.VectorSubcoreMesh(core_axis_name="c", subcore_axis_name="s"),
        scratch_types=[
            pltpu.VMEM((_N_CODES * _CODE_DIM,), jnp.float32),
            pltpu.VMEM((_TOK_PER_W,), jnp.int32),
            pltpu.VMEM((2, _CODE_DIM, _CHUNK), jnp.float32),
            pltpu.SemaphoreType.DMA,
        ],
        compiler_params=pltpu.CompilerParams(needs_layout_passes=False),
    )
    zq = sc_lookup(codebook.reshape(-1), idx3.reshape(_NW, _TOK_PER_W))

    z_q_st = zq.reshape(B, C, H, W)
    indices = idx3.reshape(B, H * W)
    vq_loss = acc[0, 0] * jnp.float32(1.25 / (B * C * H * W))
    return z_q_st, vq_loss, indices


# speedup vs baseline: 1.1786x; 1.1786x over previous
import functools

import jax
import jax.numpy as jnp
from jax import lax
from jax.experimental import pallas as pl
from jax.experimental.pallas import tpu as pltpu
from jax.experimental.pallas import tpu_sc as plsc

_N_CODES = 512
_CODE_DIM = 64
_H_TILE = 16

_NC = 2
_NS = 16
_NW = _NC * _NS
_TOK_PER_W = 2048
_CHUNK = 256


def _vq_tc_body(z_ref, cb_ref, idx_ref, acc_ref):
    x = z_ref[0].reshape(_CODE_DIM, _H_TILE * 64)
    cb = cb_ref[...]
    cnorm = jnp.sum(cb * cb, axis=1)
    znorm = jnp.sum(x * x, axis=0)
    s2 = lax.dot_general(cb * (-2.0), x, (((1,), (0,)), ((), ())),
                         preferred_element_type=jnp.float32)
    dist = (znorm[None, :] + s2) + cnorm[:, None]
    m = jnp.min(dist, axis=0)
    kiota = lax.broadcasted_iota(jnp.int32, dist.shape, 0)
    idx = jnp.min(jnp.where(dist == m[None, :], kiota, _N_CODES), axis=0)
    idx_ref[0, 0, :] = idx

    @pl.when((pl.program_id(0) == 0) & (pl.program_id(1) == 0))
    def _():
        acc_ref[0, 0] = 0.0

    acc_ref[0, 0] += jnp.sum(m)


def _sc_lookup_body(cb_hbm, idx_hbm, out_hbm, cb_v, idx_v, buf_v, sem):
    w = lax.axis_index("s") * _NC + lax.axis_index("c")
    b = w // 2
    hw0 = (w % 2) * _TOK_PER_W
    pltpu.sync_copy(cb_hbm, cb_v)
    pltpu.sync_copy(idx_hbm.at[w], idx_v)

    n_chunks = _TOK_PER_W // _CHUNK

    def outer(o, carry):
        for k in range(2):
            ch = o * 2 + k

            @pl.when(o > 0)
            def _():
                pltpu.make_async_copy(
                    buf_v.at[k], out_hbm.at[b, :, pl.ds(0, _CHUNK)], sem
                ).wait()

            @plsc.parallel_loop(0, _CHUNK // 16, unroll=2)
            def _inner(l):
                i16 = idx_v[pl.ds(ch * _CHUNK + l * 16, 16)]
                base = i16 * _CODE_DIM
                for c in range(_CODE_DIM):
                    vals = plsc.load_gather(cb_v, [base + c])
                    buf_v[k, c, pl.ds(l * 16, 16)] = vals
            pltpu.async_copy(
                buf_v.at[k],
                out_hbm.at[b, :, pl.ds(hw0 + ch * _CHUNK, _CHUNK)],
                sem,
            )
        return carry

    lax.fori_loop(0, n_chunks // 2, outer, 0)
    for k in range(2):
        pltpu.make_async_copy(
            buf_v.at[k], out_hbm.at[b, :, pl.ds(0, _CHUNK)], sem
        ).wait()


@jax.jit
def kernel(z, codebook):
    B, C, H, W = z.shape
    nh = H // _H_TILE
    idx3, acc = pl.pallas_call(
        _vq_tc_body,
        grid=(B, nh),
        in_specs=[
            pl.BlockSpec((1, C, _H_TILE, W), lambda b, h: (b, 0, h, 0)),
            pl.BlockSpec((_N_CODES, _CODE_DIM), lambda b, h: (0, 0)),
        ],
        out_specs=[
            pl.BlockSpec((1, 1, _H_TILE * W), lambda b, h, nh=nh: (b * nh + h, 0, 0)),
            pl.BlockSpec(memory_space=pltpu.SMEM),
        ],
        out_shape=[
            jax.ShapeDtypeStruct((B * nh, 1, _H_TILE * W), jnp.int32),
            jax.ShapeDtypeStruct((1, 1), jnp.float32),
        ],
    )(z, codebook)

    sc_lookup = pl.kernel(
        _sc_lookup_body,
        out_type=jax.ShapeDtypeStruct((B, C, H * W), jnp.float32),
        mesh=plsc.VectorSubcoreMesh(core_axis_name="c", subcore_axis_name="s"),
        scratch_types=[
            pltpu.VMEM((_N_CODES * _CODE_DIM,), jnp.float32),
            pltpu.VMEM((_TOK_PER_W,), jnp.int32),
            pltpu.VMEM((2, _CODE_DIM, _CHUNK), jnp.float32),
            pltpu.SemaphoreType.DMA,
        ],
        compiler_params=pltpu.CompilerParams(needs_layout_passes=False),
    )
    zq = sc_lookup(codebook.reshape(-1), idx3.reshape(_NW, _TOK_PER_W))

    z_q_st = zq.reshape(B, C, H, W)
    indices = idx3.reshape(B, H * W)
    vq_loss = acc[0, 0] * jnp.float32(1.25 / (B * C * H * W))
    return z_q_st, vq_loss, indices


# speedup vs baseline: 1.4204x; 1.2052x over previous
import functools

import jax
import jax.numpy as jnp
from jax import lax
from jax.experimental import pallas as pl
from jax.experimental.pallas import tpu as pltpu
from jax.experimental.pallas import tpu_sc as plsc

_N_CODES = 512
_CODE_DIM = 64
_H_TILE = 16

_NC = 2
_NS = 16
_NW = _NC * _NS
_CHUNK = 128


def _vq_tc_body(z_ref, cb_ref, idx_ref, acc_ref):
    x = z_ref[0].reshape(_CODE_DIM, _H_TILE * 64)
    cb = cb_ref[...]
    cnorm = jnp.sum(cb * cb, axis=1)
    znorm = jnp.sum(x * x, axis=0)
    s2 = lax.dot_general(cb * (-2.0), x, (((1,), (0,)), ((), ())),
                         preferred_element_type=jnp.float32)
    dist = (znorm[None, :] + s2) + cnorm[:, None]
    m = jnp.min(dist, axis=0)
    kiota = lax.broadcasted_iota(jnp.int32, dist.shape, 0)
    idx = jnp.min(jnp.where(dist == m[None, :], kiota, _N_CODES), axis=0)
    idx_ref[0, 0, :] = idx

    @pl.when((pl.program_id(0) == 0) & (pl.program_id(1) == 0))
    def _():
        acc_ref[0, 0] = 0.0

    acc_ref[0, 0] += jnp.sum(m)


def _sc_gather_body(cb_hbm, idx_hbm, out_hbm, idx_v, rows_v, sem):
    w = lax.axis_index("s") * _NC + lax.axis_index("c")
    pltpu.sync_copy(idx_hbm.at[w], idx_v)
    for half in range(2):
        descs = []
        for j in range(8):
            jj = half * 8 + j
            descs.append(pltpu.async_copy(
                cb_hbm.at[idx_v.at[jj]], rows_v.at[pl.ds(j * _CHUNK, _CHUNK)],
                sem))
        for d in descs:
            d.wait()
        pltpu.sync_copy(rows_v, out_hbm.at[w, pl.ds(half * 1024, 1024)])


@jax.jit
def kernel(z, codebook):
    B, C, H, W = z.shape
    nh = H // _H_TILE
    n_tok = B * H * W
    idx3, acc = pl.pallas_call(
        _vq_tc_body,
        grid=(B, nh),
        in_specs=[
            pl.BlockSpec((1, C, _H_TILE, W), lambda b, h: (b, 0, h, 0)),
            pl.BlockSpec((_N_CODES, _CODE_DIM), lambda b, h: (0, 0)),
        ],
        out_specs=[
            pl.BlockSpec((1, 1, _H_TILE * W), lambda b, h, nh=nh: (b * nh + h, 0, 0)),
            pl.BlockSpec(memory_space=pltpu.SMEM),
        ],
        out_shape=[
            jax.ShapeDtypeStruct((B * nh, 1, _H_TILE * W), jnp.int32),
            jax.ShapeDtypeStruct((1, 1), jnp.float32),
        ],
    )(z, codebook)

    per_w = n_tok // _NW
    sc_gather = pl.kernel(
        _sc_gather_body,
        out_type=jax.ShapeDtypeStruct((_NW, per_w, _CODE_DIM), jnp.float32),
        mesh=plsc.VectorSubcoreMesh(core_axis_name="c", subcore_axis_name="s"),
        scratch_types=[
            pltpu.VMEM((per_w // _CHUNK, _CHUNK), jnp.int32),
            pltpu.VMEM((per_w // 2, _CODE_DIM), jnp.float32),
            pltpu.SemaphoreType.DMA,
        ],
        compiler_params=pltpu.CompilerParams(use_tc_tiling_on_sc=False),
    )
    zq_rows = sc_gather(codebook, idx3.reshape(_NW, per_w // _CHUNK, _CHUNK))

    z_q_st = zq_rows.reshape(B, H, W, C).transpose(0, 3, 1, 2)
    indices = idx3.reshape(B, H * W)
    vq_loss = acc[0, 0] * jnp.float32(1.25 / (B * C * H * W))
    return z_q_st, vq_loss, indices
